# baseline (speedup 1.0000x reference)
import functools

import jax
import jax.numpy as jnp
from jax import lax
from jax.experimental import pallas as pl
from jax.experimental.pallas import tpu as pltpu
from jax.experimental.pallas import tpu_sc as plsc

B = 16384
EMB = 64
NFEAT = 4
OUT_D = NFEAT * EMB

_info = plsc.get_sparse_core_info()
_NC, _NS = _info.num_cores, _info.num_subcores
_NW = _NC * _NS
_BPW = B // _NW
_CH = 128
_NCH = _BPW // _CH
_NSTEP = NFEAT * _NCH
_NBUF = 2


def _body(idx_hbm, w0, w1, w2, w3, out_hbm, idx_v, gbuf, gsem0, gsem1,
          osem0, osem1):
    wid = lax.axis_index("s") * _NC + lax.axis_index("c")
    base = wid * _BPW

    pltpu.sync_copy(idx_hbm.at[pl.ds(wid * _NSTEP, _NSTEP)], idx_v)

    tables = (w0, w1, w2, w3)
    gsems = (gsem0, gsem1)
    osems = (osem0, osem1)

    def fire(step):
        feat, _ = divmod(step, _NCH)
        return pltpu.async_copy(
            tables[feat].at[idx_v.at[step]],
            gbuf.at[step % _NBUF],
            gsems[step % _NBUF],
        )

    def drain(step):
        feat, chunk = divmod(step, _NCH)
        return pltpu.async_copy(
            gbuf.at[step % _NBUF],
            out_hbm.at[feat, pl.ds(base + chunk * _CH, _CH)],
            osems[step % _NBUF],
        )

    pending = {s: fire(s) for s in range(_NBUF)}
    writes = {}
    for s in range(_NSTEP):
        pending.pop(s).wait()
        writes[s] = drain(s)
        if s + _NBUF < _NSTEP:
            writes.pop(s).wait()
            pending[s + _NBUF] = fire(s + _NBUF)
    for s in sorted(writes):
        writes.pop(s).wait()


@jax.jit
def _encode(idx_arr, w0, w1, w2, w3):
    mesh = plsc.VectorSubcoreMesh(core_axis_name="c", subcore_axis_name="s")
    k = functools.partial(
        pl.kernel,
        mesh=mesh,
        out_type=jax.ShapeDtypeStruct((NFEAT, B, 2 * EMB), jnp.float32),
        scratch_types=[
            pltpu.VMEM((_NSTEP, _CH), jnp.int32),
            pltpu.VMEM((_NBUF, _CH, 2 * EMB), jnp.float32),
            pltpu.SemaphoreType.DMA,
            pltpu.SemaphoreType.DMA,
            pltpu.SemaphoreType.DMA,
            pltpu.SemaphoreType.DMA,
        ],
    )(_body)
    return k(idx_arr, w0, w1, w2, w3)


def kernel(X_cat, W_zipcode, W_category, W_brand, W_platform):
    idx_arr = (X_cat.T.reshape(NFEAT, _NW, _NCH, _CH)
               .transpose(1, 0, 2, 3).reshape(_NW * _NSTEP, _CH))
    pads = [jnp.pad(w, ((0, 0), (0, EMB)))
            for w in (W_zipcode, W_category, W_brand, W_platform)]
    planes = _encode(idx_arr, *pads)
    return planes[..., :EMB].transpose(1, 0, 2).reshape(B, OUT_D)


# speedup vs baseline: 1.3549x; 1.3549x over previous
import functools

import jax
import jax.numpy as jnp
from jax import lax
from jax.experimental import pallas as pl
from jax.experimental.pallas import tpu as pltpu
from jax.experimental.pallas import tpu_sc as plsc

B = 16384
EMB = 64
NFEAT = 4
OUT_D = NFEAT * EMB

_info = plsc.get_sparse_core_info()
_NC, _NS = _info.num_cores, _info.num_subcores
_NW = _NC * _NS
_BPW = B // _NW
_IPW = NFEAT * _BPW
_CH = 128
_NCH = _BPW // _CH
_NSTEP = NFEAT * _NCH
_NBUF = 2


def _body(idx_hbm, w0, w1, w2, w3, out_hbm, idx_v, gbuf, gsem0, gsem1,
          osem0, osem1):
    wid = lax.axis_index("s") * _NC + lax.axis_index("c")
    base = wid * _BPW

    pltpu.sync_copy(idx_hbm.at[pl.ds(wid * _IPW, _IPW)], idx_v)

    tables = (w0, w1, w2, w3)
    gsems = (gsem0, gsem1)
    osems = (osem0, osem1)

    def fire(step):
        feat, _ = divmod(step, _NCH)
        tab = tables[feat]
        buf = step % _NBUF
        sem = gsems[buf]

        def group(g, _):
            vec = idx_v[pl.ds(step * _CH + g * 16, 16)]
            for l in range(16):
                pltpu.async_copy(
                    tab.at[pl.ds(vec[l], 1)],
                    gbuf.at[buf, pl.ds(g * 16 + l, 1)],
                    sem,
                )
            return 0

        lax.fori_loop(0, _CH // 16, group, 0, unroll=False)

    def drain(step):
        buf = step % _NBUF
        pltpu.make_async_copy(
            tables[0].at[pl.ds(0, _CH)], gbuf.at[buf], gsems[buf]
        ).wait()

    def write(step):
        feat, chunk = divmod(step, _NCH)
        buf = step % _NBUF
        return pltpu.async_copy(
            gbuf.at[buf],
            out_hbm.at[feat, pl.ds(base + chunk * _CH, _CH)],
            osems[buf],
        )

    fire(0)
    fire(1)
    writes = {}
    for s in range(_NSTEP):
        drain(s)
        writes[s] = write(s)
        if s + _NBUF < _NSTEP:
            writes.pop(s).wait()
            fire(s + _NBUF)
    for s in sorted(writes):
        writes.pop(s).wait()


@jax.jit
def _encode(idx_arr, w0, w1, w2, w3):
    mesh = plsc.VectorSubcoreMesh(core_axis_name="c", subcore_axis_name="s")
    k = functools.partial(
        pl.kernel,
        mesh=mesh,
        out_type=jax.ShapeDtypeStruct((NFEAT, B, EMB), jnp.float32),
        scratch_types=[
            pltpu.VMEM((_IPW,), jnp.int32),
            pltpu.VMEM((_NBUF, _CH, EMB), jnp.float32),
            pltpu.SemaphoreType.DMA,
            pltpu.SemaphoreType.DMA,
            pltpu.SemaphoreType.DMA,
            pltpu.SemaphoreType.DMA,
        ],
    )(_body)
    return k(idx_arr, w0, w1, w2, w3)


def kernel(X_cat, W_zipcode, W_category, W_brand, W_platform):
    idx_arr = (X_cat.T.reshape(NFEAT, _NW, _BPW)
               .transpose(1, 0, 2).reshape(_NW * _IPW))
    planes = _encode(idx_arr, W_zipcode, W_category, W_brand, W_platform)
    return planes.transpose(1, 0, 2).reshape(B, OUT_D)


# speedup vs baseline: 2.1873x; 1.6144x over previous
import functools

import jax
import jax.numpy as jnp
from jax import lax
from jax.experimental import pallas as pl
from jax.experimental.pallas import tpu as pltpu
from jax.experimental.pallas import tpu_sc as plsc

B = 16384
V = 100001
EMB = 64
NFEAT = 4
OUT_D = NFEAT * EMB

_info = plsc.get_sparse_core_info()
_NC, _NS = _info.num_cores, _info.num_subcores
_NW = _NC * _NS
_DPW = NFEAT * EMB // _NW
_Q = B // 4


def _body(w0, w1, w2, w3, xT, out, idx_v, slab, obuf, osem0, osem1):
    wid = lax.axis_index("s") * _NC + lax.axis_index("c")
    f = wid // _DPW
    pltpu.sync_copy(xT.at[pl.ds(f, 1)], idx_v)

    tables = (w0, w1, w2, w3)
    zeros = jnp.zeros((16,), jnp.int32)
    osems = (osem0, osem1)
    writes = []
    for ei in range(_DPW):
        e = (wid % _DPW) * _DPW + ei
        for fi in range(NFEAT):
            @pl.when(f == fi)
            def _():
                pltpu.sync_copy(tables[fi].at[pl.ds(e, 1)], slab)
        for h in range(4):
            def grp(g, _):
                iv = idx_v[0, pl.ds(h * _Q + g * 16, 16)]
                vals = plsc.load_gather(slab, [zeros, iv])
                obuf[h % 2, 0, pl.ds(g * 16, 16)] = vals
                return 0
            lax.fori_loop(0, _Q // 16, grp, 0)
            if len(writes) >= 2:
                writes.pop(0).wait()
            writes.append(pltpu.async_copy(
                obuf.at[h % 2],
                out.at[pl.ds(f * EMB + e, 1), pl.ds(h * _Q, _Q)],
                osems[h % 2]))
    for wcp in writes:
        wcp.wait()


@jax.jit
def _encode(w0, w1, w2, w3, xT):
    mesh = plsc.VectorSubcoreMesh(core_axis_name="c", subcore_axis_name="s")
    k = functools.partial(
        pl.kernel,
        mesh=mesh,
        compiler_params=pltpu.CompilerParams(needs_layout_passes=False),
        out_type=jax.ShapeDtypeStruct((OUT_D, B), jnp.float32),
        scratch_types=[
            pltpu.VMEM((1, B), jnp.int32),
            pltpu.VMEM((1, V), jnp.float32),
            pltpu.VMEM((2, 1, _Q), jnp.float32),
            pltpu.SemaphoreType.DMA,
            pltpu.SemaphoreType.DMA,
        ],
    )(_body)
    return k(w0, w1, w2, w3, xT)


def kernel(X_cat, W_zipcode, W_category, W_brand, W_platform):
    outT = _encode(W_zipcode.T, W_category.T, W_brand.T, W_platform.T,
                   X_cat.T)
    return outT.T
